# baseline (speedup 1.0000x reference)
import functools

import jax
import jax.numpy as jnp
from jax import lax
from jax.experimental import pallas as pl
from jax.experimental.pallas import tpu as pltpu
from jax.experimental.pallas import tpu_sc as plsc

B = 4096
L = 200
E = 64
HID = 256
NOUT = 5

NC = 2
NS = 16
NW = NC * NS
BPW = B // NW

NBUF = 2
UNROLL = 8
IUNROLL = 16
LANES = 16
EV = E // LANES


V = 1000000
NFULL = V // 128


def _sc_tr_body(embT_hbm, out_hbm, in_v, out_v, in_t, out_t, si, so):
    wid = lax.axis_index("s") * NC + lax.axis_index("c")
    iotas = [lax.iota(jnp.int32, LANES) + c * LANES for c in range(EV)]

    def blk(t):
        return wid + t * NW

    def stage_start(buf, i):
        pltpu.async_copy(
            embT_hbm.at[:, pl.ds(i * 128, 128)], in_v.at[buf], si
        )

    def stage_wait(buf):
        pltpu.make_async_copy(
            embT_hbm.at[:, pl.ds(0, 128)], in_v.at[buf], si
        ).wait()

    def transpose_block(buf, i, width):
        @pl.loop(0, width)
        def _r(r):
            ridx = jnp.full((LANES,), r, jnp.int32)
            for c in range(EV):
                v = plsc.load_gather(in_v.at[buf], [iotas[c], ridx])
                out_v[buf, r, pl.ds(c * LANES, LANES)] = v
        pltpu.async_copy(
            out_v.at[buf, pl.ds(0, width), :],
            out_hbm.at[pl.ds(i * 128, width), :],
            so,
        ).wait()

    nmine = (NFULL - 1 - wid) // NW + 1

    stage_start(0, blk(0))

    @pl.loop(0, (nmine + 1) // 2)
    def _g(g):
        for sub in range(2):
            t = g * 2 + sub

            @pl.when(t < nmine)
            def _():
                stage_wait(sub)

                @pl.when(t + 1 < nmine)
                def _():
                    stage_start(1 - sub, blk(t + 1))

                transpose_block(sub, blk(t), 128)

    @pl.when(wid == NFULL % NW)
    def _tail():
        pltpu.async_copy(
            embT_hbm.at[:, pl.ds(NFULL * 128, 64)], in_t, si
        ).wait()

        @pl.loop(0, 64)
        def _r(r):
            ridx = jnp.full((LANES,), r, jnp.int32)
            for c in range(EV):
                v = plsc.load_gather(in_t, [iotas[c], ridx])
                out_t[r, pl.ds(c * LANES, LANES)] = v
        pltpu.async_copy(
            out_t, out_hbm.at[pl.ds(NFULL * 128, 64), :], so
        ).wait()


@jax.jit
def _sc_transpose(embT):
    mesh = plsc.VectorSubcoreMesh(core_axis_name="c", subcore_axis_name="s")
    f = pl.kernel(
        _sc_tr_body,
        out_type=jax.ShapeDtypeStruct((1000000, E), jnp.float32),
        mesh=mesh,
        scratch_types=[
            pltpu.VMEM((2, E, 128), jnp.float32),
            pltpu.VMEM((2, 128, E), jnp.float32),
            pltpu.VMEM((E, 64), jnp.float32),
            pltpu.VMEM((64, E), jnp.float32),
            pltpu.SemaphoreType.DMA,
            pltpu.SemaphoreType.DMA,
        ],
        compiler_params=pltpu.CompilerParams(
            use_tc_tiling_on_sc=True, needs_layout_passes=False
        ),
    )
    return f(embT)


def _sc_pool_body(idx_hbm, emb_hbm, out_hbm, idx_v, rows_v, pooled_v,
                  s0, s1):
    sems = (s0, s1)
    wid = lax.axis_index("s") * NC + lax.axis_index("c")

    pltpu.sync_copy(idx_hbm.at[pl.ds(wid * BPW * L, BPW * L)], idx_v)

    def issue(b, e_local):
        def enq(vec, u, j):
            pltpu.async_copy(
                emb_hbm.at[vec[u], :],
                rows_v.at[b, j, :],
                sems[b],
            )

        @pl.loop(0, L // IUNROLL)
        def _rows(jv):
            vec = idx_v[pl.ds(e_local * L + jv * IUNROLL, IUNROLL)]
            for u in range(IUNROLL):
                enq(vec, u, jv * IUNROLL + u)

        tail = L % IUNROLL
        if tail:
            vec = idx_v[pl.ds(e_local * L + L - IUNROLL, IUNROLL)]
            for u in range(IUNROLL - tail, IUNROLL):
                enq(vec, u, L - IUNROLL + u)

    def drain(b):
        pltpu.make_async_copy(
            emb_hbm.at[pl.ds(0, L), :], rows_v.at[b], sems[b]
        ).wait()

    for b in range(NBUF):
        issue(b, b)

    zero = jnp.zeros((LANES,), jnp.float32)
    inv_l = jnp.float32(1.0 / L)

    @pl.loop(0, BPW // NBUF)
    def _group(g):
        for b in range(NBUF):
            e = g * NBUF + b
            drain(b)

            def acc_body(jv, accs):
                accs = list(accs)
                for u in range(UNROLL):
                    j = jv * UNROLL + u
                    for c in range(EV):
                        accs[c] = accs[c] + rows_v[b, j, pl.ds(c * LANES, LANES)]
                return tuple(accs)

            accs = lax.fori_loop(0, L // UNROLL, acc_body, (zero,) * EV)
            for c in range(EV):
                pooled_v[pl.ds(e * E + c * LANES, LANES)] = accs[c] * inv_l

            nxt = e + NBUF

            @pl.when(nxt < BPW)
            def _():
                issue(b, nxt)

    pltpu.sync_copy(pooled_v, out_hbm.at[pl.ds(wid * BPW * E, BPW * E)])


@jax.jit
def _sc_pool(idx, emb):
    mesh = plsc.VectorSubcoreMesh(core_axis_name="c", subcore_axis_name="s")
    f = pl.kernel(
        _sc_pool_body,
        out_type=jax.ShapeDtypeStruct((B * E,), jnp.float32),
        mesh=mesh,
        scratch_types=[
            pltpu.VMEM((BPW * L,), jnp.int32),
            pltpu.VMEM((NBUF, L, E), jnp.float32),
            pltpu.VMEM((BPW * E,), jnp.float32),
            pltpu.SemaphoreType.DMA,
            pltpu.SemaphoreType.DMA,
        ],
        compiler_params=pltpu.CompilerParams(use_tc_tiling_on_sc=True),
    )
    return f(idx, emb)


def _mlp_body(x_ref, w1_ref, b1_ref, w2_ref, b2_ref, o_ref):
    x = x_ref[...]
    h = jnp.dot(x, w1_ref[...], preferred_element_type=jnp.float32)
    h = jnp.maximum(h + b1_ref[...], 0.0)
    logits = jnp.dot(h, w2_ref[...], preferred_element_type=jnp.float32)
    logits = logits + b2_ref[...]
    shifted = logits - jnp.max(logits, axis=-1, keepdims=True)
    lse = jnp.log(jnp.sum(jnp.exp(shifted), axis=-1, keepdims=True))
    o_ref[...] = shifted - lse


@jax.jit
def _tc_mlp(pooled, W1, b1, W2, b2):
    return pl.pallas_call(
        _mlp_body,
        out_shape=jax.ShapeDtypeStruct((B, NOUT), jnp.float32),
    )(pooled, W1, b1.reshape(1, HID), W2, b2.reshape(1, NOUT))


def kernel(input, emb, W1, b1, W2, b2):
    emb_rm = _sc_transpose(emb.T)
    pooled = _sc_pool(input.reshape(-1), emb_rm).reshape(B, E)
    return _tc_mlp(pooled, W1, b1, W2, b2)


# speedup vs baseline: 1.0903x; 1.0903x over previous
import functools

import jax
import jax.numpy as jnp
from jax import lax
from jax.experimental import pallas as pl
from jax.experimental.pallas import tpu as pltpu
from jax.experimental.pallas import tpu_sc as plsc

B = 4096
L = 200
E = 64
HID = 256
NOUT = 5

NC = 2
NS = 16
NW = NC * NS
BPW = B // NW

NBUF = 2
UNROLL = 8
IUNROLL = 16
LANES = 16
EV = E // LANES


V = 1000000
NFULL = V // 128


def _sc_tr_body(embT_hbm, out_hbm, in_v, out_v, in_t, out_t, si, so):
    wid = lax.axis_index("s") * NC + lax.axis_index("c")
    iotas = [lax.iota(jnp.int32, LANES) + c * LANES for c in range(EV)]

    def blk(t):
        return wid + t * NW

    def stage_start(buf, i):
        pltpu.async_copy(
            embT_hbm.at[:, pl.ds(i * 128, 128)], in_v.at[buf], si
        )

    def stage_wait(buf):
        pltpu.make_async_copy(
            embT_hbm.at[:, pl.ds(0, 128)], in_v.at[buf], si
        ).wait()

    def transpose_compute(buf, src):
        @pl.loop(0, 16)
        def _rv(rv):
            for ru in range(8):
                r = rv * 8 + ru
                ridx = jnp.full((LANES,), r, jnp.int32)
                for c in range(EV):
                    v = plsc.load_gather(src, [iotas[c], ridx])
                    out_v[buf, r, pl.ds(c * LANES, LANES)] = v

    def out_start(buf, i):
        pltpu.async_copy(
            out_v.at[buf], out_hbm.at[pl.ds(i * 128, 128), :], so
        )

    def out_wait(buf):
        pltpu.make_async_copy(
            out_v.at[buf], out_hbm.at[pl.ds(0, 128), :], so
        ).wait()

    nmine = (NFULL - 1 - wid) // NW + 1

    stage_start(0, blk(0))

    @pl.loop(0, (nmine + 1) // 2)
    def _g(g):
        for sub in range(2):
            t = g * 2 + sub

            @pl.when(t < nmine)
            def _():
                stage_wait(sub)

                @pl.when(t + 1 < nmine)
                def _():
                    stage_start(1 - sub, blk(t + 1))

                @pl.when(t >= 2)
                def _():
                    out_wait(sub)

                transpose_compute(sub, in_v.at[sub])
                out_start(sub, blk(t))

    for k in (2, 1):
        @pl.when(nmine >= k)
        def _():
            out_wait((nmine - k) % 2)

    @pl.when(wid == NFULL % NW)
    def _tail():
        pltpu.async_copy(
            embT_hbm.at[:, pl.ds(NFULL * 128, 64)], in_t, si
        ).wait()

        @pl.loop(0, 8)
        def _rv(rv):
            for ru in range(8):
                r = rv * 8 + ru
                ridx = jnp.full((LANES,), r, jnp.int32)
                for c in range(EV):
                    v = plsc.load_gather(in_t, [iotas[c], ridx])
                    out_t[r, pl.ds(c * LANES, LANES)] = v
        pltpu.async_copy(
            out_t, out_hbm.at[pl.ds(NFULL * 128, 64), :], so
        ).wait()


@jax.jit
def _sc_transpose(embT):
    mesh = plsc.VectorSubcoreMesh(core_axis_name="c", subcore_axis_name="s")
    f = pl.kernel(
        _sc_tr_body,
        out_type=jax.ShapeDtypeStruct((1000000, E), jnp.float32),
        mesh=mesh,
        scratch_types=[
            pltpu.VMEM((2, E, 128), jnp.float32),
            pltpu.VMEM((2, 128, E), jnp.float32),
            pltpu.VMEM((E, 64), jnp.float32),
            pltpu.VMEM((64, E), jnp.float32),
            pltpu.SemaphoreType.DMA,
            pltpu.SemaphoreType.DMA,
        ],
        compiler_params=pltpu.CompilerParams(
            use_tc_tiling_on_sc=True, needs_layout_passes=False
        ),
    )
    return f(embT)


def _sc_pool_body(idx_hbm, emb_hbm, out_hbm, idx_v, rows_v, pooled_v,
                  s0, s1):
    sems = (s0, s1)
    wid = lax.axis_index("s") * NC + lax.axis_index("c")

    pltpu.sync_copy(idx_hbm.at[pl.ds(wid * BPW * L, BPW * L)], idx_v)

    def issue(b, e_local):
        def enq(vec, u, j):
            pltpu.async_copy(
                emb_hbm.at[vec[u], :],
                rows_v.at[b, j, :],
                sems[b],
            )

        @pl.loop(0, L // IUNROLL)
        def _rows(jv):
            vec = idx_v[pl.ds(e_local * L + jv * IUNROLL, IUNROLL)]
            for u in range(IUNROLL):
                enq(vec, u, jv * IUNROLL + u)

        tail = L % IUNROLL
        if tail:
            vec = idx_v[pl.ds(e_local * L + L - IUNROLL, IUNROLL)]
            for u in range(IUNROLL - tail, IUNROLL):
                enq(vec, u, L - IUNROLL + u)

    def drain(b):
        pltpu.make_async_copy(
            emb_hbm.at[pl.ds(0, L), :], rows_v.at[b], sems[b]
        ).wait()

    for b in range(NBUF):
        issue(b, b)

    zero = jnp.zeros((LANES,), jnp.float32)
    inv_l = jnp.float32(1.0 / L)

    @pl.loop(0, BPW // NBUF)
    def _group(g):
        for b in range(NBUF):
            e = g * NBUF + b
            drain(b)

            def acc_body(jv, accs):
                accs = list(accs)
                for u in range(UNROLL):
                    j = jv * UNROLL + u
                    for c in range(EV):
                        accs[c] = accs[c] + rows_v[b, j, pl.ds(c * LANES, LANES)]
                return tuple(accs)

            accs = lax.fori_loop(0, L // UNROLL, acc_body, (zero,) * EV)
            for c in range(EV):
                pooled_v[pl.ds(e * E + c * LANES, LANES)] = accs[c] * inv_l

            nxt = e + NBUF

            @pl.when(nxt < BPW)
            def _():
                issue(b, nxt)

    pltpu.sync_copy(pooled_v, out_hbm.at[pl.ds(wid * BPW * E, BPW * E)])


@jax.jit
def _sc_pool(idx, emb):
    mesh = plsc.VectorSubcoreMesh(core_axis_name="c", subcore_axis_name="s")
    f = pl.kernel(
        _sc_pool_body,
        out_type=jax.ShapeDtypeStruct((B * E,), jnp.float32),
        mesh=mesh,
        scratch_types=[
            pltpu.VMEM((BPW * L,), jnp.int32),
            pltpu.VMEM((NBUF, L, E), jnp.float32),
            pltpu.VMEM((BPW * E,), jnp.float32),
            pltpu.SemaphoreType.DMA,
            pltpu.SemaphoreType.DMA,
        ],
        compiler_params=pltpu.CompilerParams(use_tc_tiling_on_sc=True),
    )
    return f(idx, emb)


def _mlp_body(x_ref, w1_ref, b1_ref, w2_ref, b2_ref, o_ref):
    x = x_ref[...]
    h = jnp.dot(x, w1_ref[...], preferred_element_type=jnp.float32)
    h = jnp.maximum(h + b1_ref[...], 0.0)
    logits = jnp.dot(h, w2_ref[...], preferred_element_type=jnp.float32)
    logits = logits + b2_ref[...]
    shifted = logits - jnp.max(logits, axis=-1, keepdims=True)
    lse = jnp.log(jnp.sum(jnp.exp(shifted), axis=-1, keepdims=True))
    o_ref[...] = shifted - lse


@jax.jit
def _tc_mlp(pooled, W1, b1, W2, b2):
    return pl.pallas_call(
        _mlp_body,
        out_shape=jax.ShapeDtypeStruct((B, NOUT), jnp.float32),
    )(pooled, W1, b1.reshape(1, HID), W2, b2.reshape(1, NOUT))


def kernel(input, emb, W1, b1, W2, b2):
    emb_rm = _sc_transpose(emb.T)
    pooled = _sc_pool(input.reshape(-1), emb_rm).reshape(B, E)
    return _tc_mlp(pooled, W1, b1, W2, b2)


# speedup vs baseline: 3.5310x; 3.2384x over previous
import functools

import jax
import jax.numpy as jnp
from jax import lax
from jax.experimental import pallas as pl
from jax.experimental.pallas import tpu as pltpu
from jax.experimental.pallas import tpu_sc as plsc

B = 4096
L = 200
E = 64
HID = 256
NOUT = 5

NC = 2
NS = 16
NW = NC * NS
BPW = B // NW

NBUF = 3
CHUNK = 16
NCHUNK = L // CHUNK
TAIL = L % CHUNK
LANES = 16
EV = E // LANES


def _sc_pool_body(idx_hbm, emb_hbm, out_hbm, idx_v, rows_v, pooled_v,
                  s0, s1, s2):
    sems = (s0, s1, s2)
    wid = lax.axis_index("s") * NC + lax.axis_index("c")

    pltpu.sync_copy(idx_hbm.at[pl.ds(wid * BPW * L, BPW * L)], idx_v)

    def enq(nb, vec, u, j):
        pltpu.async_copy(
            emb_hbm.at[vec[u], :],
            rows_v.at[nb, j, :],
            sems[nb],
        )

    def issue(nb, e_local):
        @pl.loop(0, NCHUNK)
        def _rows(jv):
            vec = idx_v[pl.ds(e_local * L + jv * CHUNK, CHUNK)]
            for u in range(CHUNK):
                enq(nb, vec, u, jv * CHUNK + u)
        vec = idx_v[pl.ds(e_local * L + L - CHUNK, CHUNK)]
        for u in range(CHUNK - TAIL, CHUNK):
            enq(nb, vec, u, L - CHUNK + u)

    def drain(b):
        pltpu.make_async_copy(
            emb_hbm.at[pl.ds(0, L), :], rows_v.at[b], sems[b]
        ).wait()

    zero = jnp.zeros((LANES,), jnp.float32)
    inv_l = jnp.float32(1.0 / L)

    def process(e, b, nb, do_issue):
        drain(b)

        def acc_chunk(jv, accs):
            accs = list(accs)
            if do_issue:
                vec = idx_v[pl.ds((e + 2) * L + jv * CHUNK, CHUNK)]
                for u in range(CHUNK):
                    enq(nb, vec, u, jv * CHUNK + u)
            for u in range(CHUNK):
                j = jv * CHUNK + u
                for c in range(EV):
                    accs[c] = accs[c] + rows_v[b, j, pl.ds(c * LANES, LANES)]
            return tuple(accs)

        accs = lax.fori_loop(0, NCHUNK, acc_chunk, (zero,) * EV)

        accs = list(accs)
        if do_issue:
            vec = idx_v[pl.ds((e + 2) * L + L - CHUNK, CHUNK)]
            for u in range(CHUNK - TAIL, CHUNK):
                enq(nb, vec, u, L - CHUNK + u)
        for u in range(TAIL):
            j = NCHUNK * CHUNK + u
            for c in range(EV):
                accs[c] = accs[c] + rows_v[b, j, pl.ds(c * LANES, LANES)]

        for c in range(EV):
            pooled_v[pl.ds(e * E + c * LANES, LANES)] = accs[c] * inv_l

    issue(0, 0)
    issue(1, 1)

    @pl.loop(0, (BPW - 2) // NBUF)
    def _group(g):
        for sub in range(NBUF):
            e = g * NBUF + sub
            process(e, sub % NBUF, (sub + 2) % NBUF, True)

    process(BPW - 2, (BPW - 2) % NBUF, 0, False)
    process(BPW - 1, (BPW - 1) % NBUF, 0, False)

    pltpu.sync_copy(pooled_v, out_hbm.at[pl.ds(wid * BPW * E, BPW * E)])


@jax.jit
def _sc_pool(idx, emb):
    mesh = plsc.VectorSubcoreMesh(core_axis_name="c", subcore_axis_name="s")
    f = pl.kernel(
        _sc_pool_body,
        out_type=jax.ShapeDtypeStruct((B * E,), jnp.float32),
        mesh=mesh,
        scratch_types=[
            pltpu.VMEM((BPW * L,), jnp.int32),
            pltpu.VMEM((NBUF, L, E), jnp.float32),
            pltpu.VMEM((BPW * E,), jnp.float32),
            pltpu.SemaphoreType.DMA,
            pltpu.SemaphoreType.DMA,
            pltpu.SemaphoreType.DMA,
        ],
        compiler_params=pltpu.CompilerParams(use_tc_tiling_on_sc=True),
    )
    return f(idx, emb)


def _mlp_body(x_ref, w1_ref, b1_ref, w2_ref, b2_ref, o_ref):
    x = x_ref[...]
    h = jnp.dot(x, w1_ref[...], preferred_element_type=jnp.float32)
    h = jnp.maximum(h + b1_ref[...], 0.0)
    logits = jnp.dot(h, w2_ref[...], preferred_element_type=jnp.float32)
    logits = logits + b2_ref[...]
    shifted = logits - jnp.max(logits, axis=-1, keepdims=True)
    lse = jnp.log(jnp.sum(jnp.exp(shifted), axis=-1, keepdims=True))
    o_ref[...] = shifted - lse


@jax.jit
def _tc_mlp(pooled, W1, b1, W2, b2):
    return pl.pallas_call(
        _mlp_body,
        out_shape=jax.ShapeDtypeStruct((B, NOUT), jnp.float32),
    )(pooled, W1, b1.reshape(1, HID), W2, b2.reshape(1, NOUT))


def kernel(input, emb, W1, b1, W2, b2):
    pooled = _sc_pool(input.reshape(-1), emb).reshape(B, E)
    return _tc_mlp(pooled, W1, b1, W2, b2)
